# baseline (speedup 1.0000x reference)
import functools

import jax
import jax.numpy as jnp
from jax import lax
from jax.experimental import pallas as pl
from jax.experimental.pallas import tpu as pltpu
from jax.experimental.pallas import tpu_sc as plsc

N_CODES = 1024
DIM = 64
DIM_PAD = 128
N_ROWS = 32 * 1024
BLOCK_ROWS = 2048
COMMITMENT = 0.25

_SC_CORES = 2
_SC_SUBCORES = 16
_SC_WORKERS = _SC_CORES * _SC_SUBCORES
_ROWS_PER_WORKER = N_ROWS // _SC_WORKERS
_IDX_CHUNK = 128
_N_CHUNKS = _ROWS_PER_WORKER // _IDX_CHUNK
_ROUND_CHUNKS = 4
_ROUND_ROWS = _ROUND_CHUNKS * _IDX_CHUNK


def _vq_body(x_ref, esq_ref, e_ref, idx_ref, loss_ref):
    x = x_ref[...]
    e = e_ref[...]
    e2 = e + e
    mm2 = lax.dot_general(x, e2, (((1,), (1,)), ((), ())),
                          preferred_element_type=jnp.float32)
    x_sq = jnp.sum(x * x, axis=1, keepdims=True)
    d = (x_sq + esq_ref[...]) - mm2
    min_d = jnp.min(d, axis=1, keepdims=True)
    colf = lax.broadcasted_iota(jnp.int32, d.shape, 1).astype(jnp.float32)
    idxf = jnp.min(jnp.where(d == min_d, colf, float(N_CODES)), axis=1)
    idx = idxf.astype(jnp.int32)
    idx_ref[...] = idx.reshape(BLOCK_ROWS // 128, 128)

    @pl.when(pl.program_id(0) == 0)
    def _init():
        loss_ref[0, 0] = 0.0

    loss_ref[0, 0] += jnp.sum(min_d)

    @pl.when(pl.program_id(0) == pl.num_programs(0) - 1)
    def _finish():
        loss_ref[0, 0] *= (1.0 + COMMITMENT) / (N_ROWS * DIM)


def _distance_argmin(flat, e_sq, embeddings):
    grid = N_ROWS // BLOCK_ROWS
    return pl.pallas_call(
        _vq_body,
        grid=(grid,),
        in_specs=[
            pl.BlockSpec((BLOCK_ROWS, DIM), lambda i: (i, 0)),
            pl.BlockSpec((1, N_CODES), lambda i: (0, 0)),
            pl.BlockSpec((N_CODES, DIM), lambda i: (0, 0)),
        ],
        out_specs=[
            pl.BlockSpec((BLOCK_ROWS // 128, 128), lambda i: (i, 0)),
            pl.BlockSpec((1, 1), lambda i: (0, 0), memory_space=pltpu.SMEM),
        ],
        out_shape=[
            jax.ShapeDtypeStruct((N_ROWS // 128, 128), jnp.int32),
            jax.ShapeDtypeStruct((1, 1), jnp.float32),
        ],
    )(flat, e_sq, embeddings)


@functools.lru_cache(maxsize=None)
def _make_sc_gather():
    @functools.partial(
        pl.kernel,
        out_type=jax.ShapeDtypeStruct((N_ROWS, DIM), jnp.float32),
        mesh=plsc.VectorSubcoreMesh(core_axis_name="c", subcore_axis_name="s"),
        scratch_types=[
            pltpu.VMEM((_N_CHUNKS, _IDX_CHUNK), jnp.int32),
            pltpu.VMEM((_ROWS_PER_WORKER, DIM), jnp.float32),
            pltpu.SemaphoreType.DMA((_N_CHUNKS,)),
            pltpu.SemaphoreType.DMA,
        ],
        compiler_params=pltpu.CompilerParams(use_tc_tiling_on_sc=False),
    )
    def _sc_gather(table_hbm, idx_hbm, out_hbm, idx_v, rows_v, gsem, wsem):
        wid = lax.axis_index("s") * _SC_CORES + lax.axis_index("c")
        base = wid * _ROWS_PER_WORKER
        pltpu.sync_copy(idx_hbm.at[wid], idx_v)
        gathers = [
            pltpu.async_copy(
                table_hbm.at[idx_v.at[j]],
                rows_v.at[pl.ds(j * _IDX_CHUNK, _IDX_CHUNK)],
                gsem.at[j],
            )
            for j in range(_N_CHUNKS)
        ]
        writebacks = []
        for j in range(_N_CHUNKS):
            gathers[j].wait()
            writebacks.append(pltpu.async_copy(
                rows_v.at[pl.ds(j * _IDX_CHUNK, _IDX_CHUNK)],
                out_hbm.at[pl.ds(base + j * _IDX_CHUNK, _IDX_CHUNK)],
                wsem,
            ))
        for cp in writebacks:
            cp.wait()

    return _sc_gather


def kernel(inputs, embeddings):
    flat = inputs.reshape(-1, DIM)
    e_sq = jnp.sum(embeddings ** 2, axis=1)[None, :]
    idx_packed, loss11 = _distance_argmin(flat, e_sq, embeddings)
    idx3d = idx_packed.reshape(_SC_WORKERS, _N_CHUNKS, _IDX_CHUNK)
    quantized = _make_sc_gather()(embeddings, idx3d)
    return (quantized.reshape(inputs.shape), loss11[0, 0],
            idx_packed.reshape(N_ROWS, 1))


# speedup vs baseline: 1.0267x; 1.0267x over previous
import functools

import jax
import jax.numpy as jnp
from jax import lax
from jax.experimental import pallas as pl
from jax.experimental.pallas import tpu as pltpu
from jax.experimental.pallas import tpu_sc as plsc

N_CODES = 1024
DIM = 64
DIM_PAD = 128
N_ROWS = 32 * 1024
BLOCK_ROWS = 2048
COMMITMENT = 0.25

_SC_CORES = 2
_SC_SUBCORES = 16
_SC_WORKERS = _SC_CORES * _SC_SUBCORES
_ROWS_PER_WORKER = N_ROWS // _SC_WORKERS
_IDX_CHUNK = 128
_N_CHUNKS = _ROWS_PER_WORKER // _IDX_CHUNK
_ROUND_CHUNKS = 4
_ROUND_ROWS = _ROUND_CHUNKS * _IDX_CHUNK


def _vq_body(x_ref, esq_ref, e_ref, idx_ref, loss_ref):
    x = x_ref[...]
    e = e_ref[...]
    e2 = e + e
    mm2 = lax.dot_general(x, e2, (((1,), (1,)), ((), ())),
                          preferred_element_type=jnp.float32)
    x_sq = jnp.sum(x * x, axis=1, keepdims=True)
    d = (x_sq + esq_ref[...]) - mm2
    min_d = jnp.min(d, axis=1, keepdims=True)
    colf = lax.broadcasted_iota(jnp.int32, d.shape, 1).astype(jnp.float32)
    idxf = jnp.min(jnp.where(d == min_d, colf, float(N_CODES)), axis=1)
    idx = idxf.astype(jnp.int32)
    idx_ref[...] = idx.reshape(BLOCK_ROWS // 128, 128)

    @pl.when(pl.program_id(0) == 0)
    def _init():
        loss_ref[0, 0] = 0.0

    loss_ref[0, 0] += jnp.sum(min_d)

    @pl.when(pl.program_id(0) == pl.num_programs(0) - 1)
    def _finish():
        loss_ref[0, 0] *= (1.0 + COMMITMENT) / (N_ROWS * DIM)


def _distance_argmin(flat, e_sq, embeddings):
    grid = N_ROWS // BLOCK_ROWS
    return pl.pallas_call(
        _vq_body,
        grid=(grid,),
        in_specs=[
            pl.BlockSpec((BLOCK_ROWS, DIM), lambda i: (i, 0)),
            pl.BlockSpec((1, N_CODES), lambda i: (0, 0)),
            pl.BlockSpec((N_CODES, DIM), lambda i: (0, 0)),
        ],
        out_specs=[
            pl.BlockSpec((BLOCK_ROWS // 128, 128), lambda i: (i, 0)),
            pl.BlockSpec((1, 1), lambda i: (0, 0), memory_space=pltpu.SMEM),
        ],
        out_shape=[
            jax.ShapeDtypeStruct((N_ROWS // 128, 128), jnp.int32),
            jax.ShapeDtypeStruct((1, 1), jnp.float32),
        ],
    )(flat, e_sq, embeddings)


@functools.lru_cache(maxsize=None)
def _make_sc_gather():
    _RC = 2
    _NR = _N_CHUNKS // _RC
    _RROWS = _RC * _IDX_CHUNK

    @functools.partial(
        pl.kernel,
        out_type=jax.ShapeDtypeStruct((N_ROWS, DIM_PAD), jnp.float32),
        mesh=plsc.VectorSubcoreMesh(core_axis_name="c", subcore_axis_name="s"),
        scratch_types=[
            pltpu.VMEM((_N_CHUNKS, _IDX_CHUNK), jnp.int32),
            pltpu.VMEM((2, _RROWS, DIM_PAD), jnp.float32),
            pltpu.SemaphoreType.DMA((_N_CHUNKS,)),
            pltpu.SemaphoreType.DMA((_NR,)),
        ],
    )
    def _sc_gather(table_hbm, idx_hbm, out_hbm, idx_v, buf, gsem, wsem):
        wid = lax.axis_index("s") * _SC_CORES + lax.axis_index("c")
        base = wid * _ROWS_PER_WORKER
        pltpu.sync_copy(idx_hbm.at[wid], idx_v)

        def fire_round(r):
            return [
                pltpu.async_copy(
                    table_hbm.at[idx_v.at[r * _RC + j]],
                    buf.at[r % 2, pl.ds(j * _IDX_CHUNK, _IDX_CHUNK)],
                    gsem.at[r * _RC + j],
                )
                for j in range(_RC)
            ]

        gathers = fire_round(0)
        wbs = [None] * _NR
        for r in range(_NR):
            for cp in gathers:
                cp.wait()
            wbs[r] = pltpu.async_copy(
                buf.at[r % 2],
                out_hbm.at[pl.ds(base + r * _RROWS, _RROWS)],
                wsem.at[r],
            )
            if r + 1 < _NR:
                if r >= 1:
                    wbs[r - 1].wait()
                gathers = fire_round(r + 1)
        wbs[_NR - 2].wait()
        wbs[_NR - 1].wait()

    return _sc_gather


def kernel(inputs, embeddings):
    flat = inputs.reshape(-1, DIM)
    e_sq = jnp.sum(embeddings ** 2, axis=1)[None, :]
    e_pad = jnp.pad(embeddings, ((0, 0), (0, DIM_PAD - DIM)))
    idx_packed, loss11 = _distance_argmin(flat, e_sq, embeddings)
    idx3d = idx_packed.reshape(_SC_WORKERS, _N_CHUNKS, _IDX_CHUNK)
    qwide = _make_sc_gather()(e_pad, idx3d)
    quantized = qwide[:, :DIM]
    return (quantized.reshape(inputs.shape), loss11[0, 0],
            idx_packed.reshape(N_ROWS, 1))


# speedup vs baseline: 1.0526x; 1.0252x over previous
import functools

import jax
import jax.numpy as jnp
from jax import lax
from jax.experimental import pallas as pl
from jax.experimental.pallas import tpu as pltpu
from jax.experimental.pallas import tpu_sc as plsc

N_CODES = 1024
DIM = 64
DIM_PAD = 128
N_ROWS = 32 * 1024
BLOCK_ROWS = 4096
COMMITMENT = 0.25

_SC_CORES = 2
_SC_SUBCORES = 16
_SC_WORKERS = _SC_CORES * _SC_SUBCORES
_ROWS_PER_WORKER = N_ROWS // _SC_WORKERS
_IDX_CHUNK = 128
_N_CHUNKS = _ROWS_PER_WORKER // _IDX_CHUNK
_ROUND_CHUNKS = 4
_ROUND_ROWS = _ROUND_CHUNKS * _IDX_CHUNK


def _vq_body(x_ref, esq_ref, e_ref, idx_ref, loss_ref):
    x = x_ref[...]
    e = e_ref[...]
    e2 = e + e
    mm2 = lax.dot_general(x, e2, (((1,), (1,)), ((), ())),
                          preferred_element_type=jnp.float32)
    x_sq = jnp.sum(x * x, axis=1, keepdims=True)
    d = (x_sq + esq_ref[...]) - mm2
    min_d = jnp.min(d, axis=1, keepdims=True)
    colf = lax.broadcasted_iota(jnp.int32, d.shape, 1).astype(jnp.float32)
    idxf = jnp.min(jnp.where(d == min_d, colf, float(N_CODES)), axis=1)
    idx = idxf.astype(jnp.int32)
    idx_ref[...] = idx.reshape(BLOCK_ROWS // 128, 128)

    @pl.when(pl.program_id(0) == 0)
    def _init():
        loss_ref[0, 0] = 0.0

    loss_ref[0, 0] += jnp.sum(min_d)

    @pl.when(pl.program_id(0) == pl.num_programs(0) - 1)
    def _finish():
        loss_ref[0, 0] *= (1.0 + COMMITMENT) / (N_ROWS * DIM)


def _distance_argmin(flat, e_sq, embeddings):
    grid = N_ROWS // BLOCK_ROWS
    return pl.pallas_call(
        _vq_body,
        grid=(grid,),
        in_specs=[
            pl.BlockSpec((BLOCK_ROWS, DIM), lambda i: (i, 0)),
            pl.BlockSpec((1, N_CODES), lambda i: (0, 0)),
            pl.BlockSpec((N_CODES, DIM), lambda i: (0, 0)),
        ],
        out_specs=[
            pl.BlockSpec((BLOCK_ROWS // 128, 128), lambda i: (i, 0)),
            pl.BlockSpec((1, 1), lambda i: (0, 0), memory_space=pltpu.SMEM),
        ],
        out_shape=[
            jax.ShapeDtypeStruct((N_ROWS // 128, 128), jnp.int32),
            jax.ShapeDtypeStruct((1, 1), jnp.float32),
        ],
    )(flat, e_sq, embeddings)


@functools.lru_cache(maxsize=None)
def _make_sc_gather():
    @functools.partial(
        pl.kernel,
        out_type=jax.ShapeDtypeStruct((N_ROWS, DIM_PAD), jnp.float32),
        mesh=plsc.VectorSubcoreMesh(core_axis_name="c", subcore_axis_name="s"),
        scratch_types=[
            pltpu.VMEM((_N_CHUNKS, _IDX_CHUNK), jnp.int32),
            pltpu.VMEM((_ROUND_ROWS, DIM_PAD), jnp.float32),
            pltpu.SemaphoreType.DMA,
        ],
    )
    def _sc_gather(table_hbm, idx_hbm, out_hbm, idx_v, rows_v, sem):
        wid = lax.axis_index("s") * _SC_CORES + lax.axis_index("c")
        base = wid * _ROWS_PER_WORKER
        pltpu.sync_copy(idx_hbm.at[wid], idx_v)
        for r in range(_N_CHUNKS // _ROUND_CHUNKS):
            copies = [
                pltpu.async_copy(
                    table_hbm.at[idx_v.at[r * _ROUND_CHUNKS + j]],
                    rows_v.at[pl.ds(j * _IDX_CHUNK, _IDX_CHUNK)],
                    sem,
                )
                for j in range(_ROUND_CHUNKS)
            ]
            for cp in copies:
                cp.wait()
            pltpu.sync_copy(
                rows_v,
                out_hbm.at[pl.ds(base + r * _ROUND_ROWS, _ROUND_ROWS)],
            )

    return _sc_gather


def kernel(inputs, embeddings):
    flat = inputs.reshape(-1, DIM)
    e_sq = jnp.sum(embeddings ** 2, axis=1)[None, :]
    e_pad = jnp.pad(embeddings, ((0, 0), (0, DIM_PAD - DIM)))
    idx_packed, loss11 = _distance_argmin(flat, e_sq, embeddings)
    idx3d = idx_packed.reshape(_SC_WORKERS, _N_CHUNKS, _IDX_CHUNK)
    qwide = _make_sc_gather()(e_pad, idx3d)
    quantized = qwide[:, :DIM]
    return (quantized.reshape(inputs.shape), loss11[0, 0],
            idx_packed.reshape(N_ROWS, 1))


# speedup vs baseline: 1.1303x; 1.0738x over previous
import functools

import jax
import jax.numpy as jnp
from jax import lax
from jax.experimental import pallas as pl
from jax.experimental.pallas import tpu as pltpu
from jax.experimental.pallas import tpu_sc as plsc

N_CODES = 1024
DIM = 64
DIM_PAD = 128
N_ROWS = 32 * 1024
BLOCK_ROWS = 4096
COMMITMENT = 0.25

_SC_CORES = 2
_SC_SUBCORES = 16
_SC_WORKERS = _SC_CORES * _SC_SUBCORES
_ROWS_PER_WORKER = N_ROWS // _SC_WORKERS
_IDX_CHUNK = 128
_N_CHUNKS = _ROWS_PER_WORKER // _IDX_CHUNK
_ROUND_CHUNKS = 4
_ROUND_ROWS = _ROUND_CHUNKS * _IDX_CHUNK


def _vq_body(x_ref, esq_ref, e_ref, idx_ref, loss_ref):
    x = x_ref[...]
    e = e_ref[...]
    e2 = e + e
    mm2 = lax.dot_general(x, e2, (((1,), (1,)), ((), ())),
                          preferred_element_type=jnp.float32)
    x_sq = jnp.sum(x * x, axis=1, keepdims=True)
    d = (x_sq + esq_ref[...]) - mm2
    lanes = 128
    nseg = N_CODES // lanes
    base = lax.broadcasted_iota(jnp.int32, (1, lanes), 1).astype(jnp.float32)
    pairs = [(d[:, k * lanes:(k + 1) * lanes], base + float(k * lanes))
             for k in range(nseg)]
    while len(pairs) > 1:
        nxt = []
        for i in range(0, len(pairs), 2):
            (va, ca), (vb, cb) = pairs[i], pairs[i + 1]
            take = vb < va
            nxt.append((jnp.minimum(va, vb), jnp.where(take, cb, ca)))
        pairs = nxt
    v, c = pairs[0]
    min_d = jnp.min(v, axis=1, keepdims=True)
    idxf = jnp.min(jnp.where(v == min_d, c, float(N_CODES)), axis=1)
    idx = idxf.astype(jnp.int32)
    idx_ref[...] = idx.reshape(BLOCK_ROWS // 128, 128)

    @pl.when(pl.program_id(0) == 0)
    def _init():
        loss_ref[0, 0] = 0.0

    loss_ref[0, 0] += jnp.sum(min_d)

    @pl.when(pl.program_id(0) == pl.num_programs(0) - 1)
    def _finish():
        loss_ref[0, 0] *= (1.0 + COMMITMENT) / (N_ROWS * DIM)


def _distance_argmin(flat, e_sq, embeddings):
    grid = N_ROWS // BLOCK_ROWS
    return pl.pallas_call(
        _vq_body,
        grid=(grid,),
        in_specs=[
            pl.BlockSpec((BLOCK_ROWS, DIM), lambda i: (i, 0)),
            pl.BlockSpec((1, N_CODES), lambda i: (0, 0)),
            pl.BlockSpec((N_CODES, DIM), lambda i: (0, 0)),
        ],
        out_specs=[
            pl.BlockSpec((BLOCK_ROWS // 128, 128), lambda i: (i, 0)),
            pl.BlockSpec((1, 1), lambda i: (0, 0), memory_space=pltpu.SMEM),
        ],
        out_shape=[
            jax.ShapeDtypeStruct((N_ROWS // 128, 128), jnp.int32),
            jax.ShapeDtypeStruct((1, 1), jnp.float32),
        ],
    )(flat, e_sq, embeddings)


@functools.lru_cache(maxsize=None)
def _make_sc_gather():
    @functools.partial(
        pl.kernel,
        out_type=jax.ShapeDtypeStruct((N_ROWS, DIM_PAD), jnp.float32),
        mesh=plsc.VectorSubcoreMesh(core_axis_name="c", subcore_axis_name="s"),
        scratch_types=[
            pltpu.VMEM((_N_CHUNKS, _IDX_CHUNK), jnp.int32),
            pltpu.VMEM((_ROUND_ROWS, DIM_PAD), jnp.float32),
            pltpu.SemaphoreType.DMA,
        ],
    )
    def _sc_gather(table_hbm, idx_hbm, out_hbm, idx_v, rows_v, sem):
        wid = lax.axis_index("s") * _SC_CORES + lax.axis_index("c")
        base = wid * _ROWS_PER_WORKER
        pltpu.sync_copy(idx_hbm.at[wid], idx_v)
        for r in range(_N_CHUNKS // _ROUND_CHUNKS):
            copies = [
                pltpu.async_copy(
                    table_hbm.at[idx_v.at[r * _ROUND_CHUNKS + j]],
                    rows_v.at[pl.ds(j * _IDX_CHUNK, _IDX_CHUNK)],
                    sem,
                )
                for j in range(_ROUND_CHUNKS)
            ]
            for cp in copies:
                cp.wait()
            pltpu.sync_copy(
                rows_v,
                out_hbm.at[pl.ds(base + r * _ROUND_ROWS, _ROUND_ROWS)],
            )

    return _sc_gather


def kernel(inputs, embeddings):
    flat = inputs.reshape(-1, DIM)
    e_sq = jnp.sum(embeddings ** 2, axis=1)[None, :]
    e_pad = jnp.pad(embeddings, ((0, 0), (0, DIM_PAD - DIM)))
    idx_packed, loss11 = _distance_argmin(flat, e_sq, embeddings)
    idx3d = idx_packed.reshape(_SC_WORKERS, _N_CHUNKS, _IDX_CHUNK)
    qwide = _make_sc_gather()(e_pad, idx3d)
    quantized = qwide[:, :DIM]
    return (quantized.reshape(inputs.shape), loss11[0, 0],
            idx_packed.reshape(N_ROWS, 1))
